# baseline (speedup 1.0000x reference)
import functools

import jax
import jax.numpy as jnp
from jax import lax
from jax.experimental import pallas as pl
from jax.experimental.pallas import tpu as pltpu
from jax.experimental.pallas import tpu_sc as plsc

N = 10000
E = 320000
C = 128

NC = 2
NS = 16
L = 16
NW = NC * NS
EPW = E // NW
N_PAD = 10240
SLC = N_PAD // NS

CHUNK = 40
NCH = EPW // CHUNK
RPT = N_PAD // NS
ZR = 64

NB = 5
NI = 10
LOOK = 1
ILOOK = 6

RB = 1024
GB = N_PAD // RB


def _mesh():
    return plsc.VectorSubcoreMesh(
        core_axis_name="c", subcore_axis_name="s", num_cores=NC, num_subcores=NS
    )


def _hist_body(ei_hbm, histp_hbm, ridx, cidx, hrow, hcol, mbuf, mres, stage):
    cid = lax.axis_index("c")
    sid = lax.axis_index("s")
    wid = sid * NC + cid
    base = wid * EPW
    pltpu.sync_copy(ei_hbm.at[pl.ds(base, EPW)], ridx)
    pltpu.sync_copy(ei_hbm.at[pl.ds(E + base, EPW)], cidx)

    zeros = jnp.zeros((L,), jnp.float32)

    def zbody(i, carry):
        hrow[pl.ds(i * L, L)] = zeros
        hcol[pl.ds(i * L, L)] = zeros
        return carry

    lax.fori_loop(0, N_PAD // L, zbody, 0, unroll=4)

    ones = jnp.ones((L,), jnp.float32)

    def hbody(i, carry):
        plsc.addupdate_scatter(hrow, [ridx[pl.ds(i * L, L)]], ones)
        plsc.addupdate_scatter(hcol, [cidx[pl.ds(i * L, L)]], ones)
        return carry

    lax.fori_loop(0, EPW // L, hbody, 0, unroll=4)

    pltpu.sync_copy(hrow, stage.at[sid, 0])
    pltpu.sync_copy(hcol, stage.at[sid, 1])
    plsc.subcore_barrier()

    for h in range(2):
        pltpu.sync_copy(stage.at[:, h, pl.ds(sid * SLC, SLC)], mbuf)

        def rbody(g, carry):
            acc = jnp.zeros((L,), jnp.float32)
            for j in range(NS):
                acc = acc + mbuf[j, pl.ds(g * L, L)]
            mres[pl.ds(g * L, L)] = acc
            return carry

        lax.fori_loop(0, SLC // L, rbody, 0, unroll=2)
        pltpu.sync_copy(mres, histp_hbm.at[cid, h, pl.ds(sid * SLC, SLC)])


def _agg_body(y_hbm, ei_hbm, agg_hbm, ridxb, cidxb, rows, zbuf, sagg,
              gsem, ssem, rsem, csem):
    cid = lax.axis_index("c")
    sid = lax.axis_index("s")
    base = (sid * NC + cid) * EPW

    def idx_fire(ch, ib):
        off = base + ch * CHUNK
        pltpu.async_copy(ei_hbm.at[pl.ds(off, CHUNK)], ridxb.at[ib], rsem.at[ib])
        pltpu.async_copy(ei_hbm.at[pl.ds(E + off, CHUNK)], cidxb.at[ib],
                         csem.at[ib])

    def idx_wait(ib):
        pltpu.make_async_copy(ei_hbm.at[pl.ds(0, CHUNK)], ridxb.at[ib],
                              rsem.at[ib]).wait()
        pltpu.make_async_copy(ei_hbm.at[pl.ds(0, CHUNK)], cidxb.at[ib],
                              csem.at[ib]).wait()

    def gather(ib, rb):
        pltpu.async_copy(y_hbm.at[ridxb.at[ib]], rows.at[rb], gsem.at[rb])

    def gather_wait(rb):
        pltpu.make_async_copy(y_hbm.at[ridxb.at[0]], rows.at[rb],
                              gsem.at[rb]).wait()

    def scatter(ib, rb):
        pltpu.async_copy(rows.at[rb], sagg.at[cidxb.at[ib]],
                         ssem.at[rb], add=True)

    def scatter_wait(rb):
        pltpu.make_async_copy(rows.at[rb],
                              sagg.at[cidxb.at[0]], ssem.at[rb]).wait()

    for ch in range(ILOOK):
        idx_fire(ch, ch % NI)

    zeros = jnp.zeros((L,), jnp.float32)
    cl = C // L

    def zb(i, carry):
        zbuf[i // cl, pl.ds((i % cl) * L, L)] = zeros
        return carry

    lax.fori_loop(0, ZR * cl, zb, 0, unroll=4)
    for z in range(RPT // ZR):
        pltpu.sync_copy(zbuf, sagg.at[pl.ds(sid * RPT + z * ZR, ZR)])
    plsc.subcore_barrier()

    for ch in range(LOOK):
        idx_wait(ch % NI)
        gather(ch % NI, ch % NB)

    def outer(o, carry):
        for b in range(NI):
            c = o * NI + b

            @pl.when(c >= NB - LOOK)
            def _():
                scatter_wait((b + LOOK) % NB)

            @pl.when(c + ILOOK < NCH)
            def _():
                idx_fire(c + ILOOK, (b + ILOOK) % NI)

            @pl.when(c + LOOK < NCH)
            def _():
                idx_wait((b + LOOK) % NI)
                gather((b + LOOK) % NI, (b + LOOK) % NB)

            gather_wait(b % NB)
            scatter(b, b % NB)
        return carry

    lax.fori_loop(0, NCH // NI, outer, 0)
    for c in range(NCH - NB + LOOK, NCH):
        scatter_wait(c % NB)
    plsc.subcore_barrier()
    pltpu.sync_copy(
        sagg.at[pl.ds(sid * RPT, RPT)],
        agg_hbm.at[cid, pl.ds(sid * RPT, RPT)],
    )


@functools.lru_cache(maxsize=None)
def _sc_kernels():
    mesh = _mesh()
    hist = pl.kernel(
        _hist_body,
        out_type=jax.ShapeDtypeStruct((NC, 2, N_PAD), jnp.float32),
        mesh=mesh,
        compiler_params=pltpu.CompilerParams(needs_layout_passes=False),
        scratch_types=[
            pltpu.VMEM((EPW,), jnp.int32),
            pltpu.VMEM((EPW,), jnp.int32),
            pltpu.VMEM((N_PAD,), jnp.float32),
            pltpu.VMEM((N_PAD,), jnp.float32),
            pltpu.VMEM((NS, SLC), jnp.float32),
            pltpu.VMEM((SLC,), jnp.float32),
            pltpu.VMEM_SHARED((NS, 2, N_PAD), jnp.float32),
        ],
    )
    agg = pl.kernel(
        _agg_body,
        out_type=jax.ShapeDtypeStruct((NC, N_PAD, C), jnp.float32),
        mesh=mesh,
        compiler_params=pltpu.CompilerParams(use_tc_tiling_on_sc=False),
        scratch_types=[
            pltpu.VMEM((NI, CHUNK), jnp.int32),
            pltpu.VMEM((NI, CHUNK), jnp.int32),
            pltpu.VMEM((NB, CHUNK, C), jnp.float32),
            pltpu.VMEM((ZR, C), jnp.float32),
            pltpu.VMEM_SHARED((N_PAD, C), jnp.float32),
            pltpu.SemaphoreType.DMA((NB,)),
            pltpu.SemaphoreType.DMA((NB,)),
            pltpu.SemaphoreType.DMA((NI,)),
            pltpu.SemaphoreType.DMA((NI,)),
        ],
    )
    return hist, agg


def _node_stats(hist_ref, i):
    sl = pl.ds(i * RB, RB)
    hr = hist_ref[0, 0, sl] + hist_ref[1, 0, sl]
    hc = hist_ref[0, 1, sl] + hist_ref[1, 1, sl]
    iso = jnp.where(hr + hc == 0.0, 1.0, 0.0)
    d = hc + iso + 1.0
    return iso, lax.rsqrt(d)


def _prep_body(hist_ref, x_ref, w_ref, y_ref):
    i = pl.program_id(0)
    _, dinv = _node_stats(hist_ref, i)
    xw = jnp.dot(x_ref[...], w_ref[...], preferred_element_type=jnp.float32)
    y_ref[...] = xw * dinv[:, None]


def _fin_body(aggp_ref, y_ref, hist_ref, b_ref, x_ref, gamma_ref, beta_ref,
              o_ref, tbuf, acc):
    p = pl.program_id(0)
    j = pl.program_id(1)

    @pl.when((p == 0) & (j == 0))
    def _():
        acc[...] = jnp.zeros_like(acc)

    @pl.when(p == 0)
    def _():
        iso, dinv = _node_stats(hist_ref, j)
        coeff = (1.0 + iso) * dinv
        t = ((aggp_ref[0] + aggp_ref[1]) * dinv[:, None]
             + y_ref[...] * coeff[:, None] + b_ref[...] + x_ref[...])
        tbuf[pl.ds(j * RB, RB), :] = t
        rows = j * RB + lax.broadcasted_iota(jnp.int32, (RB, 1), 0)
        tm = jnp.where(rows < N, t, 0.0)
        acc[0:1, :] = acc[0:1, :] + jnp.sum(tm, axis=0, keepdims=True)
        acc[1:2, :] = acc[1:2, :] + jnp.sum(tm * tm, axis=0, keepdims=True)

    @pl.when(p == 1)
    def _():
        mean = acc[0:1, :] / N
        var = acc[1:2, :] / N - mean * mean
        scale = lax.rsqrt(var + 1e-5) * gamma_ref[...]
        o_ref[...] = jnp.maximum(
            (tbuf[pl.ds(j * RB, RB), :] - mean) * scale + beta_ref[...], 0.0)


def kernel(x, edge_index, W, b, gamma, beta):
    eif = edge_index.astype(jnp.int32).reshape(2 * E)
    _hist_kernel, _agg_kernel = _sc_kernels()
    histp = _hist_kernel(eif)

    b2 = b.reshape(1, C)
    g2 = gamma.reshape(1, C)
    be2 = beta.reshape(1, C)

    y = pl.pallas_call(
        _prep_body,
        grid=(GB,),
        in_specs=[
            pl.BlockSpec((NC, 2, N_PAD), lambda i: (0, 0, 0)),
            pl.BlockSpec((RB, C), lambda i: (i, 0)),
            pl.BlockSpec((C, C), lambda i: (0, 0)),
        ],
        out_specs=pl.BlockSpec((RB, C), lambda i: (i, 0)),
        out_shape=jax.ShapeDtypeStruct((N_PAD, C), jnp.float32),
    )(histp, x, W)

    aggp = _agg_kernel(y, eif)

    out = pl.pallas_call(
        _fin_body,
        grid=(2, GB),
        in_specs=[
            pl.BlockSpec((NC, RB, C), lambda p, j: (0, j * (1 - p), 0)),
            pl.BlockSpec((RB, C), lambda p, j: (j * (1 - p), 0)),
            pl.BlockSpec((NC, 2, N_PAD), lambda p, j: (0, 0, 0)),
            pl.BlockSpec((1, C), lambda p, j: (0, 0)),
            pl.BlockSpec((RB, C), lambda p, j: (j * (1 - p), 0)),
            pl.BlockSpec((1, C), lambda p, j: (0, 0)),
            pl.BlockSpec((1, C), lambda p, j: (0, 0)),
        ],
        out_specs=pl.BlockSpec((RB, C), lambda p, j: (j * p, 0)),
        out_shape=jax.ShapeDtypeStruct((N, C), jnp.float32),
        scratch_shapes=[
            pltpu.VMEM((N_PAD, C), jnp.float32),
            pltpu.VMEM((2, C), jnp.float32),
        ],
    )(aggp, y, histp, b2, x, g2, be2)
    return out


# speedup vs baseline: 1.1862x; 1.1862x over previous
import functools

import jax
import jax.numpy as jnp
from jax import lax
from jax.experimental import pallas as pl
from jax.experimental.pallas import tpu as pltpu
from jax.experimental.pallas import tpu_sc as plsc

N = 10000
E = 320000
C = 128

NC = 2
NS = 16
L = 16
NW = NC * NS
EPW = E // NW
N_PAD = 10240
SLC = N_PAD // NS

CHUNK = 40
NCH = EPW // CHUNK
RPT = N_PAD // NS
ZR = 64

NB = 5
NI = 10
LOOK = 2
ILOOK = 7

RB = 1024
GB = N_PAD // RB


def _mesh():
    return plsc.VectorSubcoreMesh(
        core_axis_name="c", subcore_axis_name="s", num_cores=NC, num_subcores=NS
    )


def _hist_body(ei_hbm, histp_hbm, ridx, cidx, hrow, hcol, mbuf, mres, stage):
    cid = lax.axis_index("c")
    sid = lax.axis_index("s")
    wid = sid * NC + cid
    base = wid * EPW
    pltpu.sync_copy(ei_hbm.at[pl.ds(base, EPW)], ridx)
    pltpu.sync_copy(ei_hbm.at[pl.ds(E + base, EPW)], cidx)

    zeros = jnp.zeros((L,), jnp.float32)

    def zbody(i, carry):
        hrow[pl.ds(i * L, L)] = zeros
        hcol[pl.ds(i * L, L)] = zeros
        return carry

    lax.fori_loop(0, N_PAD // L, zbody, 0, unroll=4)

    ones = jnp.ones((L,), jnp.float32)

    def hbody(i, carry):
        plsc.addupdate_scatter(hrow, [ridx[pl.ds(i * L, L)]], ones)
        plsc.addupdate_scatter(hcol, [cidx[pl.ds(i * L, L)]], ones)
        return carry

    lax.fori_loop(0, EPW // L, hbody, 0, unroll=4)

    pltpu.sync_copy(hrow, stage.at[sid, 0])
    pltpu.sync_copy(hcol, stage.at[sid, 1])
    plsc.subcore_barrier()

    for h in range(2):
        pltpu.sync_copy(stage.at[:, h, pl.ds(sid * SLC, SLC)], mbuf)

        def rbody(g, carry):
            acc = jnp.zeros((L,), jnp.float32)
            for j in range(NS):
                acc = acc + mbuf[j, pl.ds(g * L, L)]
            mres[pl.ds(g * L, L)] = acc
            return carry

        lax.fori_loop(0, SLC // L, rbody, 0, unroll=2)
        pltpu.sync_copy(mres, histp_hbm.at[cid, h, pl.ds(sid * SLC, SLC)])


def _agg_body(y_hbm, ei_hbm, agg_hbm, ridxb, cidxb, rows, zbuf, sagg,
              gsem, ssem, rsem, csem):
    cid = lax.axis_index("c")
    sid = lax.axis_index("s")
    base = (sid * NC + cid) * EPW

    def idx_fire(ch, ib):
        off = base + ch * CHUNK
        pltpu.async_copy(ei_hbm.at[pl.ds(off, CHUNK)], ridxb.at[ib], rsem.at[ib])
        pltpu.async_copy(ei_hbm.at[pl.ds(E + off, CHUNK)], cidxb.at[ib],
                         csem.at[ib])

    def idx_wait(ib):
        pltpu.make_async_copy(ei_hbm.at[pl.ds(0, CHUNK)], ridxb.at[ib],
                              rsem.at[ib]).wait()
        pltpu.make_async_copy(ei_hbm.at[pl.ds(0, CHUNK)], cidxb.at[ib],
                              csem.at[ib]).wait()

    def gather(ib, rb):
        pltpu.async_copy(y_hbm.at[ridxb.at[ib]], rows.at[rb], gsem.at[rb])

    def gather_wait(rb):
        pltpu.make_async_copy(y_hbm.at[ridxb.at[0]], rows.at[rb],
                              gsem.at[rb]).wait()

    def scatter(ib, rb):
        pltpu.async_copy(rows.at[rb], sagg.at[cidxb.at[ib]],
                         ssem.at[rb], add=True)

    def scatter_wait(rb):
        pltpu.make_async_copy(rows.at[rb],
                              sagg.at[cidxb.at[0]], ssem.at[rb]).wait()

    for ch in range(ILOOK):
        idx_fire(ch, ch % NI)

    zeros = jnp.zeros((L,), jnp.float32)
    cl = C // L

    def zb(i, carry):
        zbuf[i // cl, pl.ds((i % cl) * L, L)] = zeros
        return carry

    lax.fori_loop(0, ZR * cl, zb, 0, unroll=4)
    for z in range(RPT // ZR):
        pltpu.sync_copy(zbuf, sagg.at[pl.ds(sid * RPT + z * ZR, ZR)])
    plsc.subcore_barrier()

    for ch in range(LOOK):
        idx_wait(ch % NI)
        gather(ch % NI, ch % NB)

    def outer(o, carry):
        for b in range(NI):
            c = o * NI + b

            @pl.when(c >= NB - LOOK)
            def _():
                scatter_wait((b + LOOK) % NB)

            @pl.when(c + ILOOK < NCH)
            def _():
                idx_fire(c + ILOOK, (b + ILOOK) % NI)

            @pl.when(c + LOOK < NCH)
            def _():
                idx_wait((b + LOOK) % NI)
                gather((b + LOOK) % NI, (b + LOOK) % NB)

            gather_wait(b % NB)
            scatter(b, b % NB)
        return carry

    lax.fori_loop(0, NCH // NI, outer, 0)
    for c in range(NCH - NB + LOOK, NCH):
        scatter_wait(c % NB)
    plsc.subcore_barrier()
    pltpu.sync_copy(
        sagg.at[pl.ds(sid * RPT, RPT)],
        agg_hbm.at[cid, pl.ds(sid * RPT, RPT)],
    )


@functools.lru_cache(maxsize=None)
def _sc_kernels():
    mesh = _mesh()
    hist = pl.kernel(
        _hist_body,
        out_type=jax.ShapeDtypeStruct((NC, 2, N_PAD), jnp.float32),
        mesh=mesh,
        compiler_params=pltpu.CompilerParams(needs_layout_passes=False),
        scratch_types=[
            pltpu.VMEM((EPW,), jnp.int32),
            pltpu.VMEM((EPW,), jnp.int32),
            pltpu.VMEM((N_PAD,), jnp.float32),
            pltpu.VMEM((N_PAD,), jnp.float32),
            pltpu.VMEM((NS, SLC), jnp.float32),
            pltpu.VMEM((SLC,), jnp.float32),
            pltpu.VMEM_SHARED((NS, 2, N_PAD), jnp.float32),
        ],
    )
    agg = pl.kernel(
        _agg_body,
        out_type=jax.ShapeDtypeStruct((NC, N_PAD, C), jnp.float32),
        mesh=mesh,
        compiler_params=pltpu.CompilerParams(use_tc_tiling_on_sc=False),
        scratch_types=[
            pltpu.VMEM((NI, CHUNK), jnp.int32),
            pltpu.VMEM((NI, CHUNK), jnp.int32),
            pltpu.VMEM((NB, CHUNK, C), jnp.float32),
            pltpu.VMEM((ZR, C), jnp.float32),
            pltpu.VMEM_SHARED((N_PAD, C), jnp.float32),
            pltpu.SemaphoreType.DMA((NB,)),
            pltpu.SemaphoreType.DMA((NB,)),
            pltpu.SemaphoreType.DMA((NI,)),
            pltpu.SemaphoreType.DMA((NI,)),
        ],
    )
    return hist, agg


def _node_stats(hist_ref, i):
    sl = pl.ds(i * RB, RB)
    hr = hist_ref[0, 0, sl] + hist_ref[1, 0, sl]
    hc = hist_ref[0, 1, sl] + hist_ref[1, 1, sl]
    iso = jnp.where(hr + hc == 0.0, 1.0, 0.0)
    d = hc + iso + 1.0
    return iso, lax.rsqrt(d)


def _mm_body(x_ref, w_ref, xw_ref):
    xw_ref[...] = jnp.dot(x_ref[...], w_ref[...],
                          preferred_element_type=jnp.float32)


def _scale_body(hist_ref, xw_ref, y_ref):
    i = pl.program_id(0)
    _, dinv = _node_stats(hist_ref, i)
    y_ref[...] = xw_ref[...] * dinv[:, None]


def _fin_body(aggp_ref, y_ref, hist_ref, b_ref, x_ref, gamma_ref, beta_ref,
              o_ref, tbuf, acc):
    p = pl.program_id(0)
    j = pl.program_id(1)

    @pl.when((p == 0) & (j == 0))
    def _():
        acc[...] = jnp.zeros_like(acc)

    @pl.when(p == 0)
    def _():
        iso, dinv = _node_stats(hist_ref, j)
        coeff = (1.0 + iso) * dinv
        t = ((aggp_ref[0] + aggp_ref[1]) * dinv[:, None]
             + y_ref[...] * coeff[:, None] + b_ref[...] + x_ref[...])
        tbuf[pl.ds(j * RB, RB), :] = t
        rows = j * RB + lax.broadcasted_iota(jnp.int32, (RB, 1), 0)
        tm = jnp.where(rows < N, t, 0.0)
        acc[0:1, :] = acc[0:1, :] + jnp.sum(tm, axis=0, keepdims=True)
        acc[1:2, :] = acc[1:2, :] + jnp.sum(tm * tm, axis=0, keepdims=True)

    @pl.when(p == 1)
    def _():
        mean = acc[0:1, :] / N
        var = acc[1:2, :] / N - mean * mean
        scale = lax.rsqrt(var + 1e-5) * gamma_ref[...]
        o_ref[...] = jnp.maximum(
            (tbuf[pl.ds(j * RB, RB), :] - mean) * scale + beta_ref[...], 0.0)


def kernel(x, edge_index, W, b, gamma, beta):
    eif = edge_index.astype(jnp.int32).reshape(2 * E)
    _hist_kernel, _agg_kernel = _sc_kernels()
    histp = _hist_kernel(eif)

    b2 = b.reshape(1, C)
    g2 = gamma.reshape(1, C)
    be2 = beta.reshape(1, C)

    xw = pl.pallas_call(
        _mm_body,
        grid=(GB,),
        in_specs=[
            pl.BlockSpec((RB, C), lambda i: (i, 0)),
            pl.BlockSpec((C, C), lambda i: (0, 0)),
        ],
        out_specs=pl.BlockSpec((RB, C), lambda i: (i, 0)),
        out_shape=jax.ShapeDtypeStruct((N_PAD, C), jnp.float32),
    )(x, W)

    y = pl.pallas_call(
        _scale_body,
        grid=(GB,),
        in_specs=[
            pl.BlockSpec((NC, 2, N_PAD), lambda i: (0, 0, 0)),
            pl.BlockSpec((RB, C), lambda i: (i, 0)),
        ],
        out_specs=pl.BlockSpec((RB, C), lambda i: (i, 0)),
        out_shape=jax.ShapeDtypeStruct((N_PAD, C), jnp.float32),
    )(histp, xw)

    aggp = _agg_kernel(y, eif)

    out = pl.pallas_call(
        _fin_body,
        grid=(2, GB),
        in_specs=[
            pl.BlockSpec((NC, RB, C), lambda p, j: (0, j * (1 - p), 0)),
            pl.BlockSpec((RB, C), lambda p, j: (j * (1 - p), 0)),
            pl.BlockSpec((NC, 2, N_PAD), lambda p, j: (0, 0, 0)),
            pl.BlockSpec((1, C), lambda p, j: (0, 0)),
            pl.BlockSpec((RB, C), lambda p, j: (j * (1 - p), 0)),
            pl.BlockSpec((1, C), lambda p, j: (0, 0)),
            pl.BlockSpec((1, C), lambda p, j: (0, 0)),
        ],
        out_specs=pl.BlockSpec((RB, C), lambda p, j: (j * p, 0)),
        out_shape=jax.ShapeDtypeStruct((N, C), jnp.float32),
        scratch_shapes=[
            pltpu.VMEM((N_PAD, C), jnp.float32),
            pltpu.VMEM((2, C), jnp.float32),
        ],
    )(aggp, y, histp, b2, x, g2, be2)
    return out


# speedup vs baseline: 1.2307x; 1.0375x over previous
import functools

import jax
import jax.numpy as jnp
from jax import lax
from jax.experimental import pallas as pl
from jax.experimental.pallas import tpu as pltpu
from jax.experimental.pallas import tpu_sc as plsc

N = 10000
E = 320000
C = 128

NC = 2
NS = 16
L = 16
NW = NC * NS
EPW = E // NW
N_PAD = 10240
SLC = N_PAD // NS

CHUNK = 40
NCH = EPW // CHUNK
RPT = N_PAD // NS
ZR = 64

NB = 5
NI = 10
LOOK = 2
ILOOK = 7

RB = 2048
GB = N_PAD // RB


def _mesh():
    return plsc.VectorSubcoreMesh(
        core_axis_name="c", subcore_axis_name="s", num_cores=NC, num_subcores=NS
    )


def _hist_body(ei_hbm, histp_hbm, ridx, cidx, hrow, hcol, mbuf, mres, stage):
    cid = lax.axis_index("c")
    sid = lax.axis_index("s")
    wid = sid * NC + cid
    base = wid * EPW
    pltpu.sync_copy(ei_hbm.at[pl.ds(base, EPW)], ridx)
    pltpu.sync_copy(ei_hbm.at[pl.ds(E + base, EPW)], cidx)

    zeros = jnp.zeros((L,), jnp.float32)

    def zbody(i, carry):
        hrow[pl.ds(i * L, L)] = zeros
        hcol[pl.ds(i * L, L)] = zeros
        return carry

    lax.fori_loop(0, N_PAD // L, zbody, 0, unroll=4)

    ones = jnp.ones((L,), jnp.float32)

    def hbody(i, carry):
        plsc.addupdate_scatter(hrow, [ridx[pl.ds(i * L, L)]], ones)
        plsc.addupdate_scatter(hcol, [cidx[pl.ds(i * L, L)]], ones)
        return carry

    lax.fori_loop(0, EPW // L, hbody, 0, unroll=4)

    pltpu.sync_copy(hrow, stage.at[sid, 0])
    pltpu.sync_copy(hcol, stage.at[sid, 1])
    plsc.subcore_barrier()

    for h in range(2):
        pltpu.sync_copy(stage.at[:, h, pl.ds(sid * SLC, SLC)], mbuf)

        def rbody(g, carry):
            acc = jnp.zeros((L,), jnp.float32)
            for j in range(NS):
                acc = acc + mbuf[j, pl.ds(g * L, L)]
            mres[pl.ds(g * L, L)] = acc
            return carry

        lax.fori_loop(0, SLC // L, rbody, 0, unroll=2)
        pltpu.sync_copy(mres, histp_hbm.at[cid, h, pl.ds(sid * SLC, SLC)])


def _agg_body(y_hbm, ei_hbm, agg_hbm, ridxb, cidxb, rows, zbuf, sagg,
              gsem, ssem, rsem, csem):
    cid = lax.axis_index("c")
    sid = lax.axis_index("s")
    base = (sid * NC + cid) * EPW

    def idx_fire(ch, ib):
        off = base + ch * CHUNK
        pltpu.async_copy(ei_hbm.at[pl.ds(off, CHUNK)], ridxb.at[ib], rsem.at[ib])
        pltpu.async_copy(ei_hbm.at[pl.ds(E + off, CHUNK)], cidxb.at[ib],
                         csem.at[ib])

    def idx_wait(ib):
        pltpu.make_async_copy(ei_hbm.at[pl.ds(0, CHUNK)], ridxb.at[ib],
                              rsem.at[ib]).wait()
        pltpu.make_async_copy(ei_hbm.at[pl.ds(0, CHUNK)], cidxb.at[ib],
                              csem.at[ib]).wait()

    def gather(ib, rb):
        pltpu.async_copy(y_hbm.at[ridxb.at[ib]], rows.at[rb], gsem.at[rb])

    def gather_wait(rb):
        pltpu.make_async_copy(y_hbm.at[ridxb.at[0]], rows.at[rb],
                              gsem.at[rb]).wait()

    def scatter(ib, rb):
        pltpu.async_copy(rows.at[rb], sagg.at[cidxb.at[ib]],
                         ssem.at[rb], add=True)

    def scatter_wait(rb):
        pltpu.make_async_copy(rows.at[rb],
                              sagg.at[cidxb.at[0]], ssem.at[rb]).wait()

    for ch in range(ILOOK):
        idx_fire(ch, ch % NI)

    zeros = jnp.zeros((L,), jnp.float32)
    cl = C // L

    def zb(i, carry):
        zbuf[i // cl, pl.ds((i % cl) * L, L)] = zeros
        return carry

    lax.fori_loop(0, ZR * cl, zb, 0, unroll=4)
    for z in range(RPT // ZR):
        pltpu.sync_copy(zbuf, sagg.at[pl.ds(sid * RPT + z * ZR, ZR)])
    plsc.subcore_barrier()

    for ch in range(LOOK):
        idx_wait(ch % NI)
        gather(ch % NI, ch % NB)

    def outer(o, carry):
        for b in range(NI):
            c = o * NI + b

            @pl.when(c >= NB - LOOK)
            def _():
                scatter_wait((b + LOOK) % NB)

            @pl.when(c + ILOOK < NCH)
            def _():
                idx_fire(c + ILOOK, (b + ILOOK) % NI)

            @pl.when(c + LOOK < NCH)
            def _():
                idx_wait((b + LOOK) % NI)
                gather((b + LOOK) % NI, (b + LOOK) % NB)

            gather_wait(b % NB)
            scatter(b, b % NB)
        return carry

    lax.fori_loop(0, NCH // NI, outer, 0)
    for c in range(NCH - NB + LOOK, NCH):
        scatter_wait(c % NB)
    plsc.subcore_barrier()
    pltpu.sync_copy(
        sagg.at[pl.ds(sid * RPT, RPT)],
        agg_hbm.at[cid, pl.ds(sid * RPT, RPT)],
    )


@functools.lru_cache(maxsize=None)
def _sc_kernels():
    mesh = _mesh()
    hist = pl.kernel(
        _hist_body,
        out_type=jax.ShapeDtypeStruct((NC, 2, N_PAD), jnp.float32),
        mesh=mesh,
        compiler_params=pltpu.CompilerParams(needs_layout_passes=False),
        scratch_types=[
            pltpu.VMEM((EPW,), jnp.int32),
            pltpu.VMEM((EPW,), jnp.int32),
            pltpu.VMEM((N_PAD,), jnp.float32),
            pltpu.VMEM((N_PAD,), jnp.float32),
            pltpu.VMEM((NS, SLC), jnp.float32),
            pltpu.VMEM((SLC,), jnp.float32),
            pltpu.VMEM_SHARED((NS, 2, N_PAD), jnp.float32),
        ],
    )
    agg = pl.kernel(
        _agg_body,
        out_type=jax.ShapeDtypeStruct((NC, N_PAD, C), jnp.float32),
        mesh=mesh,
        compiler_params=pltpu.CompilerParams(use_tc_tiling_on_sc=False),
        scratch_types=[
            pltpu.VMEM((NI, CHUNK), jnp.int32),
            pltpu.VMEM((NI, CHUNK), jnp.int32),
            pltpu.VMEM((NB, CHUNK, C), jnp.float32),
            pltpu.VMEM((ZR, C), jnp.float32),
            pltpu.VMEM_SHARED((N_PAD, C), jnp.float32),
            pltpu.SemaphoreType.DMA((NB,)),
            pltpu.SemaphoreType.DMA((NB,)),
            pltpu.SemaphoreType.DMA((NI,)),
            pltpu.SemaphoreType.DMA((NI,)),
        ],
    )
    return hist, agg


def _node_stats(hist_ref, i):
    sl = pl.ds(i * RB, RB)
    hr = hist_ref[0, 0, sl] + hist_ref[1, 0, sl]
    hc = hist_ref[0, 1, sl] + hist_ref[1, 1, sl]
    iso = jnp.where(hr + hc == 0.0, 1.0, 0.0)
    d = hc + iso + 1.0
    return iso, lax.rsqrt(d)


def _prep_body(hist_ref, x_ref, w_ref, y_ref):
    i = pl.program_id(0)
    _, dinv = _node_stats(hist_ref, i)
    xw = jnp.dot(x_ref[...], w_ref[...], preferred_element_type=jnp.float32)
    y_ref[...] = xw * dinv[:, None]


def _fin_body(aggp_ref, y_ref, hist_ref, b_ref, x_ref, gamma_ref, beta_ref,
              o_ref, tbuf, acc):
    p = pl.program_id(0)
    j = pl.program_id(1)

    @pl.when((p == 0) & (j == 0))
    def _():
        acc[...] = jnp.zeros_like(acc)

    @pl.when(p == 0)
    def _():
        iso, dinv = _node_stats(hist_ref, j)
        coeff = (1.0 + iso) * dinv
        t = ((aggp_ref[0] + aggp_ref[1]) * dinv[:, None]
             + y_ref[...] * coeff[:, None] + b_ref[...] + x_ref[...])
        tbuf[pl.ds(j * RB, RB), :] = t
        rows = j * RB + lax.broadcasted_iota(jnp.int32, (RB, 1), 0)
        tm = jnp.where(rows < N, t, 0.0)
        acc[0:1, :] = acc[0:1, :] + jnp.sum(tm, axis=0, keepdims=True)
        acc[1:2, :] = acc[1:2, :] + jnp.sum(tm * tm, axis=0, keepdims=True)

    @pl.when(p == 1)
    def _():
        mean = acc[0:1, :] / N
        var = acc[1:2, :] / N - mean * mean
        scale = lax.rsqrt(var + 1e-5) * gamma_ref[...]
        o_ref[...] = jnp.maximum(
            (tbuf[pl.ds(j * RB, RB), :] - mean) * scale + beta_ref[...], 0.0)


def kernel(x, edge_index, W, b, gamma, beta):
    eif = edge_index.astype(jnp.int32).reshape(2 * E)
    _hist_kernel, _agg_kernel = _sc_kernels()
    histp = _hist_kernel(eif)

    b2 = b.reshape(1, C)
    g2 = gamma.reshape(1, C)
    be2 = beta.reshape(1, C)

    y = pl.pallas_call(
        _prep_body,
        grid=(GB,),
        in_specs=[
            pl.BlockSpec((NC, 2, N_PAD), lambda i: (0, 0, 0)),
            pl.BlockSpec((RB, C), lambda i: (i, 0)),
            pl.BlockSpec((C, C), lambda i: (0, 0)),
        ],
        out_specs=pl.BlockSpec((RB, C), lambda i: (i, 0)),
        out_shape=jax.ShapeDtypeStruct((N_PAD, C), jnp.float32),
    )(histp, x, W)

    aggp = _agg_kernel(y, eif)

    out = pl.pallas_call(
        _fin_body,
        grid=(2, GB),
        in_specs=[
            pl.BlockSpec((NC, RB, C), lambda p, j: (0, j * (1 - p), 0)),
            pl.BlockSpec((RB, C), lambda p, j: (j * (1 - p), 0)),
            pl.BlockSpec((NC, 2, N_PAD), lambda p, j: (0, 0, 0)),
            pl.BlockSpec((1, C), lambda p, j: (0, 0)),
            pl.BlockSpec((RB, C), lambda p, j: (j * (1 - p), 0)),
            pl.BlockSpec((1, C), lambda p, j: (0, 0)),
            pl.BlockSpec((1, C), lambda p, j: (0, 0)),
        ],
        out_specs=pl.BlockSpec((RB, C), lambda p, j: (j * p, 0)),
        out_shape=jax.ShapeDtypeStruct((N, C), jnp.float32),
        scratch_shapes=[
            pltpu.VMEM((N_PAD, C), jnp.float32),
            pltpu.VMEM((2, C), jnp.float32),
        ],
    )(aggp, y, histp, b2, x, g2, be2)
    return out
